# baseline (speedup 1.0000x reference)
import functools

import jax
import jax.numpy as jnp
from jax import lax
from jax.experimental import pallas as pl
from jax.experimental.pallas import tpu as pltpu
from jax.experimental.pallas import tpu_sc as plsc

F = 128
N_RBF = 20
CUTOFF = 5.0

NC = 2
NS = 16
NW = NC * NS
B_EDGE = 80


def _mlp_body(s_ref, w1_ref, b1_ref, w2_ref, b2_ref, lo_ref, hi_ref):
    s = s_ref[...]
    h = jnp.dot(s, w1_ref[...], preferred_element_type=jnp.float32) + b1_ref[...]
    h = h * jax.nn.sigmoid(h)
    phi = jnp.dot(h, w2_ref[...], preferred_element_type=jnp.float32) + b2_ref[...]
    lo_ref[...] = phi[:, :F]
    hi_ref[...] = phi[:, F:]


def _node_mlp(s_j, W1, b1, W2, b2):
    n = s_j.shape[0]
    bn = 2000
    return pl.pallas_call(
        _mlp_body,
        grid=(n // bn,),
        in_specs=[
            pl.BlockSpec((bn, F), lambda i: (i, 0)),
            pl.BlockSpec((F, F), lambda i: (0, 0)),
            pl.BlockSpec((1, F), lambda i: (0, 0)),
            pl.BlockSpec((F, 2 * F), lambda i: (0, 0)),
            pl.BlockSpec((1, 2 * F), lambda i: (0, 0)),
        ],
        out_specs=[
            pl.BlockSpec((bn, F), lambda i: (i, 0)),
            pl.BlockSpec((bn, F), lambda i: (i, 0)),
        ],
        out_shape=[jax.ShapeDtypeStruct((n, F), jnp.float32)] * 2,
    )(s_j, W1, b1.reshape(1, F), W2, b2.reshape(1, 2 * F))


def _edge_body(r_ref, wr_ref, br_ref, cs_ref, cv0_ref, cv1_ref, cv2_ref):
    r = r_ref[...]
    d2 = jnp.sum(r * r + 1e-8, axis=1, keepdims=True)
    dist = jnp.sqrt(d2)
    unit = r / dist
    n = lax.broadcasted_iota(jnp.int32, (1, N_RBF), 1).astype(jnp.float32) + 1.0
    coef = n * (jnp.pi / CUTOFF)
    denom = jnp.where(dist == 0.0, 1.0, dist)
    rbf = jnp.where(dist >= CUTOFF, 0.0, jnp.sin(coef * dist) / denom)
    env = jnp.where(dist <= CUTOFF,
                    0.5 * (jnp.cos(jnp.pi * dist / CUTOFF) + 1.0), 0.0)
    w = jnp.dot(rbf, wr_ref[...], preferred_element_type=jnp.float32)
    w = (w + br_ref[...]) * env
    w_lo = w[:, :F]
    cs_ref[...] = _pack_pairs(w[:, F:])
    cv0_ref[...] = _pack_pairs(w_lo * unit[:, 0:1])
    cv1_ref[...] = _pack_pairs(w_lo * unit[:, 1:2])
    cv2_ref[...] = _pack_pairs(w_lo * unit[:, 2:3])


def _pack_pairs(c):
    cu = jax.lax.bitcast_convert_type(c.astype(jnp.bfloat16), jnp.uint16)
    cu = cu.reshape(c.shape[0] // 2, 2, c.shape[1]).astype(jnp.uint32)
    return cu[:, 0, :] | (cu[:, 1, :] << 16)


def _edge_coeffs(r_ij, Wr, br):
    e = r_ij.shape[0]
    be = 2000
    return pl.pallas_call(
        _edge_body,
        grid=(e // be,),
        in_specs=[
            pl.BlockSpec((be, 3), lambda i: (i, 0)),
            pl.BlockSpec((N_RBF, 2 * F), lambda i: (0, 0)),
            pl.BlockSpec((1, 2 * F), lambda i: (0, 0)),
        ],
        out_specs=[pl.BlockSpec((be // 2, F), lambda i: (i, 0))] * 4,
        out_shape=[jax.ShapeDtypeStruct((e // 2, F), jnp.uint32)] * 4,
    )(r_ij, Wr, br.reshape(1, 2 * F))


def _make_sc_pass(n, e):
    epw = e // NW
    B = B_EDGE
    nb = epw // B
    stripe = 640
    chunks = stripe // B
    mesh = plsc.VectorSubcoreMesh(core_axis_name="c", subcore_axis_name="s")

    @functools.partial(
        pl.kernel,
        mesh=mesh,
        out_type=jax.ShapeDtypeStruct((NC, n, F), jnp.float32),
        scratch_types=[
            pltpu.VMEM((2, B), jnp.int32),
            pltpu.VMEM((2, B), jnp.int32),
            pltpu.VMEM((2, B, F), jnp.float32),
            pltpu.VMEM((2, B // 2, F), jnp.uint32),
            pltpu.VMEM_SHARED((n, F), jnp.float32),
            pltpu.SemaphoreType.DMA,
            pltpu.SemaphoreType.DMA,
            pltpu.SemaphoreType.DMA,
            pltpu.SemaphoreType.DMA,
            pltpu.SemaphoreType.DMA,
            pltpu.SemaphoreType.DMA,
        ],
    )
    def sc_pass(phi_hbm, c_hbm, src_hbm, dst_hbm, out_hbm,
                src_v, dst_v, rows_v, c_v, slab, g0, g1, c0, c1, i0, i1):
        cid = lax.axis_index("c")
        sid = lax.axis_index("s")
        wid = sid * NC + cid
        row0 = sid * stripe
        my_rows = jnp.where(sid == NS - 1, n - (NS - 1) * stripe, stripe)
        gsem = (g0, g1)
        csem = (c0, c1)
        isem = (i0, i1)

        def zbody(i, carry):
            for j in range(8):
                rows_v[0, i, pl.ds(j * 16, 16)] = jnp.zeros((16,), jnp.float32)
            return carry
        lax.fori_loop(0, B, zbody, 0)
        for q in range(chunks):
            @pl.when(q * B < my_rows)
            def _():
                pltpu.sync_copy(rows_v.at[0],
                                slab.at[pl.ds(row0 + q * B, B), :])
        plsc.subcore_barrier()

        def issue_idx(g, slot):
            base = wid * epw + g * B
            pltpu.async_copy(src_hbm.at[pl.ds(base, B)], src_v.at[slot],
                             isem[slot])
            pltpu.async_copy(dst_hbm.at[pl.ds(base, B)], dst_v.at[slot],
                             isem[slot])

        def wait_idx(g, slot):
            base = wid * epw + g * B
            pltpu.make_async_copy(src_hbm.at[pl.ds(base, B)], src_v.at[slot],
                                  isem[slot]).wait()
            pltpu.make_async_copy(dst_hbm.at[pl.ds(base, B)], dst_v.at[slot],
                                  isem[slot]).wait()

        def issue_data(g, slot):
            pltpu.async_copy(phi_hbm.at[src_v.at[slot]], rows_v.at[slot],
                             gsem[slot])
            cbase = wid * (epw // 2) + g * (B // 2)
            pltpu.async_copy(c_hbm.at[pl.ds(cbase, B // 2), :], c_v.at[slot],
                             csem[slot])

        issue_idx(0, 0)
        wait_idx(0, 0)
        issue_idx(1, 1)
        issue_data(0, 0)

        def pair(i, carry):
            for b in range(2):
                g = 2 * i + b
                slot = b
                nslot = 1 - b

                @pl.when(g < nb)
                def _():
                    pltpu.make_async_copy(phi_hbm.at[src_v.at[slot]],
                                          rows_v.at[slot], gsem[slot]).wait()

                    @pl.when(g + 1 < nb)
                    def _():
                        wait_idx(g + 1, nslot)
                        issue_data(g + 1, nslot)

                    cbase = wid * (epw // 2) + g * (B // 2)
                    pltpu.make_async_copy(c_hbm.at[pl.ds(cbase, B // 2), :],
                                          c_v.at[slot], csem[slot]).wait()

                    mask = jnp.full((16,), 0xFFFF0000, jnp.uint32)

                    def mul(ep, c2):
                        for j in range(8):
                            sl = pl.ds(j * 16, 16)
                            cw = c_v[slot, ep, sl]
                            lo = lax.bitcast_convert_type(cw << 16, jnp.float32)
                            hi = lax.bitcast_convert_type(cw & mask, jnp.float32)
                            rows_v[slot, 2 * ep, sl] = (
                                rows_v[slot, 2 * ep, sl] * lo)
                            rows_v[slot, 2 * ep + 1, sl] = (
                                rows_v[slot, 2 * ep + 1, sl] * hi)
                        return c2
                    lax.fori_loop(0, B // 2, mul, 0)
                    pltpu.sync_copy(rows_v.at[slot], slab.at[dst_v.at[slot]],
                                    add=True)

                    @pl.when(g + 2 < nb)
                    def _():
                        issue_idx(g + 2, slot)
            return carry
        lax.fori_loop(0, (nb + 1) // 2, pair, 0)

        plsc.subcore_barrier()
        for q in range(chunks):
            @pl.when(q * B < my_rows)
            def _():
                pltpu.sync_copy(slab.at[pl.ds(row0 + q * B, B), :],
                                out_hbm.at[cid, pl.ds(row0 + q * B, B), :])

    return sc_pass


def _sum_body(ps_ref, p0_ref, p1_ref, p2_ref, s_ref, v0_ref, v1_ref, v2_ref):
    s_ref[...] = ps_ref[0] + ps_ref[1]
    v0_ref[...] = p0_ref[0] + p0_ref[1]
    v1_ref[...] = p1_ref[0] + p1_ref[1]
    v2_ref[...] = p2_ref[0] + p2_ref[1]


def _sum_partials(ps, p0, p1, p2):
    n = ps.shape[1]
    bn = 2000
    return pl.pallas_call(
        _sum_body,
        grid=(n // bn,),
        in_specs=[pl.BlockSpec((NC, bn, F), lambda i: (0, i, 0))] * 4,
        out_specs=[pl.BlockSpec((bn, F), lambda i: (i, 0))] * 4,
        out_shape=[jax.ShapeDtypeStruct((n, F), jnp.float32)] * 4,
    )(ps, p0, p1, p2)


def kernel(s_j, v_j, r_ij, nbrs, W1, b1, W2, b2, Wr, br):
    del v_j
    n = s_j.shape[0]
    e = r_ij.shape[0]

    phi_lo, phi_hi = _node_mlp(s_j, W1, b1, W2, b2)
    c_s, c_v0, c_v1, c_v2 = _edge_coeffs(r_ij, Wr, br)

    dst = nbrs[:, 0].astype(jnp.int32)
    src = nbrs[:, 1].astype(jnp.int32)

    sc_pass = _make_sc_pass(n, e)
    ps = sc_pass(phi_hi, c_s, src, dst)
    pv0 = sc_pass(phi_lo, c_v0, src, dst)
    pv1 = sc_pass(phi_lo, c_v1, src, dst)
    pv2 = sc_pass(phi_lo, c_v2, src, dst)

    delta_s, v0, v1, v2 = _sum_partials(ps, pv0, pv1, pv2)
    delta_v = jnp.stack([v0, v1, v2], axis=-1)
    return (delta_s, delta_v)


# speedup vs baseline: 2.5714x; 2.5714x over previous
import functools

import jax
import jax.numpy as jnp
from jax import lax
from jax.experimental import pallas as pl
from jax.experimental.pallas import tpu as pltpu
from jax.experimental.pallas import tpu_sc as plsc

F = 128
N_RBF = 20
CUTOFF = 5.0

NC = 2
NS = 16
NW = NC * NS
B_EDGE = 80


def _mlp_body(s_ref, w1_ref, b1_ref, w2_ref, b2_ref, lo_ref, hi_ref):
    s = s_ref[...]
    h = jnp.dot(s, w1_ref[...], preferred_element_type=jnp.float32) + b1_ref[...]
    h = h * jax.nn.sigmoid(h)
    phi = jnp.dot(h, w2_ref[...], preferred_element_type=jnp.float32) + b2_ref[...]
    lo_ref[...] = phi[:, :F]
    hi_ref[...] = phi[:, F:]


def _node_mlp(s_j, W1, b1, W2, b2):
    n = s_j.shape[0]
    bn = 2000
    return pl.pallas_call(
        _mlp_body,
        grid=(n // bn,),
        in_specs=[
            pl.BlockSpec((bn, F), lambda i: (i, 0)),
            pl.BlockSpec((F, F), lambda i: (0, 0)),
            pl.BlockSpec((1, F), lambda i: (0, 0)),
            pl.BlockSpec((F, 2 * F), lambda i: (0, 0)),
            pl.BlockSpec((1, 2 * F), lambda i: (0, 0)),
        ],
        out_specs=[
            pl.BlockSpec((bn, F), lambda i: (i, 0)),
            pl.BlockSpec((bn, F), lambda i: (i, 0)),
        ],
        out_shape=[jax.ShapeDtypeStruct((n, F), jnp.float32)] * 2,
    )(s_j, W1, b1.reshape(1, F), W2, b2.reshape(1, 2 * F))


def _edge_body(r_ref, wr_ref, br_ref, cs_ref, cv0_ref, cv1_ref, cv2_ref):
    r = r_ref[...]
    d2 = jnp.sum(r * r + 1e-8, axis=1, keepdims=True)
    dist = jnp.sqrt(d2)
    unit = r / dist
    ni = lax.broadcasted_iota(jnp.int32, (1, N_RBF + 1), 1)
    nf = ni.astype(jnp.float32) + 1.0
    coef = jnp.where(ni < N_RBF, nf * (jnp.pi / CUTOFF),
                     jnp.pi / (2.0 * CUTOFF))
    s = _fast_sin(dist * coef)
    denom = jnp.where(dist == 0.0, 1.0, dist)
    rbf = jnp.where(dist >= CUTOFF, 0.0, s[:, :N_RBF] / denom)
    sh = s[:, N_RBF:N_RBF + 1]
    env = jnp.where(dist <= CUTOFF, 1.0 - sh * sh, 0.0)
    w = jnp.dot(rbf, wr_ref[...], preferred_element_type=jnp.float32)
    w = (w + br_ref[...]) * env
    w_lo = w[:, :F]
    cs_ref[...] = _pack_half(w[:, F:])
    cv0_ref[...] = _pack_half(w_lo * unit[:, 0:1])
    cv1_ref[...] = _pack_half(w_lo * unit[:, 1:2])
    cv2_ref[...] = _pack_half(w_lo * unit[:, 2:3])


_SIN_C = (0.9999999959621529, -0.1666666504215155, 0.008333314504480197,
          -0.00019840310898311397, 2.753228835635207e-06,
          -2.470157612483037e-08, 1.3533146825395193e-10)


def _fast_sin(x):
    k = jnp.round(x * (1.0 / (2.0 * jnp.pi)))
    y = x - k * (2.0 * jnp.pi)
    t = y * y
    acc = jnp.float32(_SIN_C[6])
    for c in _SIN_C[5::-1]:
        acc = acc * t + jnp.float32(c)
    return y * acc


def _pack_half(c):
    half = c.shape[0] // 2
    bits = lax.bitcast_convert_type(c, jnp.uint32)
    lo = (bits[:half] + jnp.uint32(0x8000)) >> 16
    hi = (bits[half:] + jnp.uint32(0x8000)) & jnp.uint32(0xFFFF0000)
    return lo | hi


def _edge_coeffs(r_ij, Wr, br):
    e = r_ij.shape[0]
    be = 2000
    return pl.pallas_call(
        _edge_body,
        grid=(e // be,),
        in_specs=[
            pl.BlockSpec((be, 3), lambda i: (i, 0)),
            pl.BlockSpec((N_RBF, 2 * F), lambda i: (0, 0)),
            pl.BlockSpec((1, 2 * F), lambda i: (0, 0)),
        ],
        out_specs=[pl.BlockSpec((be // 2, F), lambda i: (i, 0))] * 4,
        out_shape=[jax.ShapeDtypeStruct((e // 2, F), jnp.uint32)] * 4,
    )(r_ij, Wr, br.reshape(1, 2 * F))


def _make_sc_pass(n, e):
    epw = e // NW
    B = B_EDGE
    nb = epw // B
    stripe = 640
    chunks = stripe // B
    mesh = plsc.VectorSubcoreMesh(core_axis_name="c", subcore_axis_name="s")

    @functools.partial(
        pl.kernel,
        mesh=mesh,
        out_type=jax.ShapeDtypeStruct((NC, n, F), jnp.float32),
        scratch_types=[
            pltpu.VMEM((2, B), jnp.int32),
            pltpu.VMEM((2, B), jnp.int32),
            pltpu.VMEM((2, B, F), jnp.float32),
            pltpu.VMEM((2, B // 2, F), jnp.uint32),
            pltpu.VMEM_SHARED((n, F), jnp.float32),
            pltpu.SemaphoreType.DMA,
            pltpu.SemaphoreType.DMA,
            pltpu.SemaphoreType.DMA,
            pltpu.SemaphoreType.DMA,
            pltpu.SemaphoreType.DMA,
            pltpu.SemaphoreType.DMA,
        ],
    )
    def sc_pass(phi_hbm, c_hbm, src_hbm, dst_hbm, out_hbm,
                src_v, dst_v, rows_v, c_v, slab, g0, g1, c0, c1, i0, i1):
        cid = lax.axis_index("c")
        sid = lax.axis_index("s")
        wid = sid * NC + cid
        row0 = sid * stripe
        my_rows = jnp.where(sid == NS - 1, n - (NS - 1) * stripe, stripe)
        gsem = (g0, g1)
        csem = (c0, c1)
        isem = (i0, i1)

        def zbody(i, carry):
            for j in range(8):
                rows_v[0, i, pl.ds(j * 16, 16)] = jnp.zeros((16,), jnp.float32)
            return carry
        lax.fori_loop(0, B, zbody, 0)
        for q in range(chunks):
            @pl.when(q * B < my_rows)
            def _():
                pltpu.sync_copy(rows_v.at[0],
                                slab.at[pl.ds(row0 + q * B, B), :])
        plsc.subcore_barrier()

        def issue_idx(g, slot):
            base = wid * epw + g * B
            pltpu.async_copy(src_hbm.at[pl.ds(base, B)], src_v.at[slot],
                             isem[slot])
            pltpu.async_copy(dst_hbm.at[pl.ds(base, B)], dst_v.at[slot],
                             isem[slot])

        def wait_idx(g, slot):
            base = wid * epw + g * B
            pltpu.make_async_copy(src_hbm.at[pl.ds(base, B)], src_v.at[slot],
                                  isem[slot]).wait()
            pltpu.make_async_copy(dst_hbm.at[pl.ds(base, B)], dst_v.at[slot],
                                  isem[slot]).wait()

        def issue_data(g, slot):
            pltpu.async_copy(phi_hbm.at[src_v.at[slot]], rows_v.at[slot],
                             gsem[slot])
            cbase = wid * (epw // 2) + g * (B // 2)
            pltpu.async_copy(c_hbm.at[pl.ds(cbase, B // 2), :], c_v.at[slot],
                             csem[slot])

        issue_idx(0, 0)
        wait_idx(0, 0)
        issue_idx(1, 1)
        issue_data(0, 0)

        def pair(i, carry):
            for b in range(2):
                g = 2 * i + b
                slot = b
                nslot = 1 - b

                @pl.when(g < nb)
                def _():
                    pltpu.make_async_copy(phi_hbm.at[src_v.at[slot]],
                                          rows_v.at[slot], gsem[slot]).wait()

                    @pl.when(g + 1 < nb)
                    def _():
                        wait_idx(g + 1, nslot)
                        issue_data(g + 1, nslot)

                    cbase = wid * (epw // 2) + g * (B // 2)
                    pltpu.make_async_copy(c_hbm.at[pl.ds(cbase, B // 2), :],
                                          c_v.at[slot], csem[slot]).wait()

                    mask = jnp.full((16,), 0xFFFF0000, jnp.uint32)

                    def mul(ep, c2):
                        for j in range(8):
                            sl = pl.ds(j * 16, 16)
                            cw = c_v[slot, ep, sl]
                            lo = lax.bitcast_convert_type(cw << 16, jnp.float32)
                            hi = lax.bitcast_convert_type(cw & mask, jnp.float32)
                            rows_v[slot, ep, sl] = rows_v[slot, ep, sl] * lo
                            rows_v[slot, B // 2 + ep, sl] = (
                                rows_v[slot, B // 2 + ep, sl] * hi)
                        return c2
                    lax.fori_loop(0, B // 2, mul, 0)
                    pltpu.sync_copy(rows_v.at[slot], slab.at[dst_v.at[slot]],
                                    add=True)

                    @pl.when(g + 2 < nb)
                    def _():
                        issue_idx(g + 2, slot)
            return carry
        lax.fori_loop(0, (nb + 1) // 2, pair, 0)

        plsc.subcore_barrier()
        for q in range(chunks):
            @pl.when(q * B < my_rows)
            def _():
                pltpu.sync_copy(slab.at[pl.ds(row0 + q * B, B), :],
                                out_hbm.at[cid, pl.ds(row0 + q * B, B), :])

    return sc_pass


def _sum_body(ps_ref, p0_ref, p1_ref, p2_ref, s_ref, v0_ref, v1_ref, v2_ref):
    s_ref[...] = ps_ref[0] + ps_ref[1]
    v0_ref[...] = p0_ref[0] + p0_ref[1]
    v1_ref[...] = p1_ref[0] + p1_ref[1]
    v2_ref[...] = p2_ref[0] + p2_ref[1]


def _sum_partials(ps, p0, p1, p2):
    n = ps.shape[1]
    bn = 2000
    return pl.pallas_call(
        _sum_body,
        grid=(n // bn,),
        in_specs=[pl.BlockSpec((NC, bn, F), lambda i: (0, i, 0))] * 4,
        out_specs=[pl.BlockSpec((bn, F), lambda i: (i, 0))] * 4,
        out_shape=[jax.ShapeDtypeStruct((n, F), jnp.float32)] * 4,
    )(ps, p0, p1, p2)


def kernel(s_j, v_j, r_ij, nbrs, W1, b1, W2, b2, Wr, br):
    del v_j
    n = s_j.shape[0]
    e = r_ij.shape[0]

    phi_lo, phi_hi = _node_mlp(s_j, W1, b1, W2, b2)
    c_s, c_v0, c_v1, c_v2 = _edge_coeffs(r_ij, Wr, br)

    def _pair_order(col):
        return (col.astype(jnp.int32)
                .reshape(e // 2000, 2, 25, 40)
                .transpose(0, 2, 1, 3)
                .reshape(-1))

    dst = _pair_order(nbrs[:, 0])
    src = _pair_order(nbrs[:, 1])

    sc_pass = _make_sc_pass(n, e)
    ps = sc_pass(phi_hi, c_s, src, dst)
    pv0 = sc_pass(phi_lo, c_v0, src, dst)
    pv1 = sc_pass(phi_lo, c_v1, src, dst)
    pv2 = sc_pass(phi_lo, c_v2, src, dst)

    delta_s, v0, v1, v2 = _sum_partials(ps, pv0, pv1, pv2)
    delta_v = jnp.stack([v0, v1, v2], axis=-1)
    return (delta_s, delta_v)


# speedup vs baseline: 2.7239x; 1.0593x over previous
import functools

import jax
import jax.numpy as jnp
from jax import lax
from jax.experimental import pallas as pl
from jax.experimental.pallas import tpu as pltpu
from jax.experimental.pallas import tpu_sc as plsc

F = 128
N_RBF = 20
CUTOFF = 5.0

NC = 2
NS = 16
NW = NC * NS
B_EDGE = 80


def _mlp_body(s_ref, w1_ref, b1_ref, w2_ref, b2_ref, lo_ref, hi_ref):
    s = s_ref[...]
    h = jnp.dot(s, w1_ref[...], preferred_element_type=jnp.float32) + b1_ref[...]
    h = h * jax.nn.sigmoid(h)
    phi = jnp.dot(h, w2_ref[...], preferred_element_type=jnp.float32) + b2_ref[...]
    lo_ref[...] = phi[:, :F]
    hi_ref[...] = phi[:, F:]


def _node_mlp(s_j, W1, b1, W2, b2):
    n = s_j.shape[0]
    bn = 2000
    return pl.pallas_call(
        _mlp_body,
        grid=(n // bn,),
        in_specs=[
            pl.BlockSpec((bn, F), lambda i: (i, 0)),
            pl.BlockSpec((F, F), lambda i: (0, 0)),
            pl.BlockSpec((1, F), lambda i: (0, 0)),
            pl.BlockSpec((F, 2 * F), lambda i: (0, 0)),
            pl.BlockSpec((1, 2 * F), lambda i: (0, 0)),
        ],
        out_specs=[
            pl.BlockSpec((bn, F), lambda i: (i, 0)),
            pl.BlockSpec((bn, F), lambda i: (i, 0)),
        ],
        out_shape=[jax.ShapeDtypeStruct((n, F), jnp.float32)] * 2,
    )(s_j, W1, b1.reshape(1, F), W2, b2.reshape(1, 2 * F))


def _edge_body(r_ref, wr_ref, br_ref, cs_ref, cv0_ref, cv1_ref, cv2_ref):
    r = r_ref[...]
    d2 = jnp.sum(r * r + 1e-8, axis=1, keepdims=True)
    dist = jnp.sqrt(d2)
    unit = r / dist
    ni = lax.broadcasted_iota(jnp.int32, (1, N_RBF + 1), 1)
    nf = ni.astype(jnp.float32) + 1.0
    coef = jnp.where(ni < N_RBF, nf * (jnp.pi / CUTOFF),
                     jnp.pi / (2.0 * CUTOFF))
    s = _fast_sin(dist * coef)
    denom = jnp.where(dist == 0.0, 1.0, dist)
    rbf = jnp.where(dist >= CUTOFF, 0.0, s[:, :N_RBF] / denom)
    sh = s[:, N_RBF:N_RBF + 1]
    env = jnp.where(dist <= CUTOFF, 1.0 - sh * sh, 0.0)
    w = jnp.dot(rbf, wr_ref[...], preferred_element_type=jnp.float32)
    w = (w + br_ref[...]) * env
    w_lo = w[:, :F]
    cs_ref[...] = _pack_half(w[:, F:])
    cv0_ref[...] = _pack_half(w_lo * unit[:, 0:1])
    cv1_ref[...] = _pack_half(w_lo * unit[:, 1:2])
    cv2_ref[...] = _pack_half(w_lo * unit[:, 2:3])


_SIN_C = (0.9999999959621529, -0.1666666504215155, 0.008333314504480197,
          -0.00019840310898311397, 2.753228835635207e-06,
          -2.470157612483037e-08, 1.3533146825395193e-10)


def _fast_sin(x):
    k = jnp.round(x * (1.0 / (2.0 * jnp.pi)))
    y = x - k * (2.0 * jnp.pi)
    t = y * y
    acc = jnp.float32(_SIN_C[6])
    for c in _SIN_C[5::-1]:
        acc = acc * t + jnp.float32(c)
    return y * acc


def _pack_half(c):
    half = c.shape[0] // 2
    bits = lax.bitcast_convert_type(c, jnp.uint32)
    lo = (bits[:half] + jnp.uint32(0x8000)) >> 16
    hi = (bits[half:] + jnp.uint32(0x8000)) & jnp.uint32(0xFFFF0000)
    return lo | hi


def _edge_coeffs(r_ij, Wr, br):
    e = r_ij.shape[0]
    be = 2000
    return pl.pallas_call(
        _edge_body,
        grid=(e // be,),
        in_specs=[
            pl.BlockSpec((be, 3), lambda i: (i, 0)),
            pl.BlockSpec((N_RBF, 2 * F), lambda i: (0, 0)),
            pl.BlockSpec((1, 2 * F), lambda i: (0, 0)),
        ],
        out_specs=[pl.BlockSpec((be // 2, F), lambda i: (i, 0))] * 4,
        out_shape=[jax.ShapeDtypeStruct((e // 2, F), jnp.uint32)] * 4,
    )(r_ij, Wr, br.reshape(1, 2 * F))


def _make_sc_pass(n, e):
    epw = e // NW
    B = B_EDGE
    nb = epw // B
    stripe = 640
    chunks = stripe // B
    mesh = plsc.VectorSubcoreMesh(core_axis_name="c", subcore_axis_name="s")

    @functools.partial(
        pl.kernel,
        mesh=mesh,
        out_type=jax.ShapeDtypeStruct((NC, n, F), jnp.float32),
        scratch_types=[
            pltpu.VMEM((2, B), jnp.int32),
            pltpu.VMEM((2, B), jnp.int32),
            pltpu.VMEM((2, B), jnp.int32),
            pltpu.VMEM((2, B, F), jnp.float32),
            pltpu.VMEM((2, B // 2, F), jnp.uint32),
            pltpu.VMEM_SHARED((n, F), jnp.float32),
            pltpu.SemaphoreType.DMA,
            pltpu.SemaphoreType.DMA,
            pltpu.SemaphoreType.DMA,
            pltpu.SemaphoreType.DMA,
            pltpu.SemaphoreType.DMA,
            pltpu.SemaphoreType.DMA,
            pltpu.SemaphoreType.DMA,
            pltpu.SemaphoreType.DMA,
        ],
    )
    def sc_pass(phi_hbm, c_hbm, src_hbm, dst_hbm, out_hbm,
                src_v, dst_v, dstx_v, rows_v, c_v, slab,
                g0, g1, c0, c1, i0, i1, s0, s1):
        cid = lax.axis_index("c")
        sid = lax.axis_index("s")
        wid = sid * NC + cid
        row0 = sid * stripe
        my_rows = jnp.where(sid == NS - 1, n - (NS - 1) * stripe, stripe)
        gsem = (g0, g1)
        csem = (c0, c1)
        isem = (i0, i1)
        ssem = (s0, s1)

        def zbody(i, carry):
            for j in range(8):
                rows_v[0, i, pl.ds(j * 16, 16)] = jnp.zeros((16,), jnp.float32)
            return carry
        lax.fori_loop(0, B, zbody, 0)
        for q in range(chunks):
            @pl.when(q * B < my_rows)
            def _():
                pltpu.sync_copy(rows_v.at[0],
                                slab.at[pl.ds(row0 + q * B, B), :])
        plsc.subcore_barrier()

        def issue_idx(g, slot):
            base = wid * epw + g * B
            pltpu.async_copy(src_hbm.at[pl.ds(base, B)], src_v.at[slot],
                             isem[slot])
            pltpu.async_copy(dst_hbm.at[pl.ds(base, B)], dst_v.at[slot],
                             isem[slot])

        def wait_idx(g, slot):
            base = wid * epw + g * B
            pltpu.make_async_copy(src_hbm.at[pl.ds(base, B)], src_v.at[slot],
                                  isem[slot]).wait()
            pltpu.make_async_copy(dst_hbm.at[pl.ds(base, B)], dst_v.at[slot],
                                  isem[slot]).wait()

        def issue_data(g, slot):
            pltpu.async_copy(phi_hbm.at[src_v.at[slot]], rows_v.at[slot],
                             gsem[slot])
            cbase = wid * (epw // 2) + g * (B // 2)
            pltpu.async_copy(c_hbm.at[pl.ds(cbase, B // 2), :], c_v.at[slot],
                             csem[slot])

        issue_idx(0, 0)
        wait_idx(0, 0)
        issue_idx(1, 1)
        issue_data(0, 0)

        def pair(i, carry):
            for b in range(2):
                g = 2 * i + b
                slot = b
                nslot = 1 - b

                @pl.when(g < nb)
                def _():
                    pltpu.make_async_copy(phi_hbm.at[src_v.at[slot]],
                                          rows_v.at[slot], gsem[slot]).wait()

                    @pl.when(g + 1 < nb)
                    def _():
                        wait_idx(g + 1, nslot)

                        @pl.when(g >= 1)
                        def _():
                            pltpu.make_async_copy(
                                rows_v.at[nslot], slab.at[dstx_v.at[nslot]],
                                ssem[nslot]).wait()
                        issue_data(g + 1, nslot)

                    cbase = wid * (epw // 2) + g * (B // 2)
                    pltpu.make_async_copy(c_hbm.at[pl.ds(cbase, B // 2), :],
                                          c_v.at[slot], csem[slot]).wait()

                    mask = jnp.full((16,), 0xFFFF0000, jnp.uint32)

                    def mul(ep, c2):
                        for j in range(8):
                            sl = pl.ds(j * 16, 16)
                            cw = c_v[slot, ep, sl]
                            lo = lax.bitcast_convert_type(cw << 16, jnp.float32)
                            hi = lax.bitcast_convert_type(cw & mask, jnp.float32)
                            rows_v[slot, ep, sl] = rows_v[slot, ep, sl] * lo
                            rows_v[slot, B // 2 + ep, sl] = (
                                rows_v[slot, B // 2 + ep, sl] * hi)
                        return c2
                    lax.fori_loop(0, B // 2, mul, 0)

                    def icpy(q, c2):
                        sl = pl.ds(q * 16, 16)
                        dstx_v[slot, sl] = dst_v[slot, sl]
                        return c2
                    lax.fori_loop(0, B // 16, icpy, 0)
                    pltpu.async_copy(rows_v.at[slot],
                                     slab.at[dstx_v.at[slot]], ssem[slot],
                                     add=True)

                    @pl.when(g + 2 < nb)
                    def _():
                        issue_idx(g + 2, slot)
            return carry
        lax.fori_loop(0, (nb + 1) // 2, pair, 0)

        for s in range(2):
            pltpu.make_async_copy(rows_v.at[s], slab.at[dstx_v.at[s]],
                                  ssem[s]).wait()

        plsc.subcore_barrier()
        for q in range(chunks):
            @pl.when(q * B < my_rows)
            def _():
                pltpu.sync_copy(slab.at[pl.ds(row0 + q * B, B), :],
                                out_hbm.at[cid, pl.ds(row0 + q * B, B), :])

    return sc_pass


def _sum_body(ps_ref, p0_ref, p1_ref, p2_ref, s_ref, v0_ref, v1_ref, v2_ref):
    s_ref[...] = ps_ref[0] + ps_ref[1]
    v0_ref[...] = p0_ref[0] + p0_ref[1]
    v1_ref[...] = p1_ref[0] + p1_ref[1]
    v2_ref[...] = p2_ref[0] + p2_ref[1]


def _sum_partials(ps, p0, p1, p2):
    n = ps.shape[1]
    bn = 2000
    return pl.pallas_call(
        _sum_body,
        grid=(n // bn,),
        in_specs=[pl.BlockSpec((NC, bn, F), lambda i: (0, i, 0))] * 4,
        out_specs=[pl.BlockSpec((bn, F), lambda i: (i, 0))] * 4,
        out_shape=[jax.ShapeDtypeStruct((n, F), jnp.float32)] * 4,
    )(ps, p0, p1, p2)


def kernel(s_j, v_j, r_ij, nbrs, W1, b1, W2, b2, Wr, br):
    del v_j
    n = s_j.shape[0]
    e = r_ij.shape[0]

    phi_lo, phi_hi = _node_mlp(s_j, W1, b1, W2, b2)
    c_s, c_v0, c_v1, c_v2 = _edge_coeffs(r_ij, Wr, br)

    def _pair_order(col):
        return (col.astype(jnp.int32)
                .reshape(e // 2000, 2, 25, 40)
                .transpose(0, 2, 1, 3)
                .reshape(-1))

    dst = _pair_order(nbrs[:, 0])
    src = _pair_order(nbrs[:, 1])

    sc_pass = _make_sc_pass(n, e)
    ps = sc_pass(phi_hi, c_s, src, dst)
    pv0 = sc_pass(phi_lo, c_v0, src, dst)
    pv1 = sc_pass(phi_lo, c_v1, src, dst)
    pv2 = sc_pass(phi_lo, c_v2, src, dst)

    delta_s, v0, v1, v2 = _sum_partials(ps, pv0, pv1, pv2)
    delta_v = jnp.stack([v0, v1, v2], axis=-1)
    return (delta_s, delta_v)


# speedup vs baseline: 2.7329x; 1.0033x over previous
import functools

import jax
import jax.numpy as jnp
from jax import lax
from jax.experimental import pallas as pl
from jax.experimental.pallas import tpu as pltpu
from jax.experimental.pallas import tpu_sc as plsc

F = 128
N_RBF = 20
CUTOFF = 5.0

NC = 2
NS = 16
NW = NC * NS
B_EDGE = 80


def _mlp_body(s_ref, w1_ref, b1_ref, w2_ref, b2_ref, lo_ref, hi_ref):
    s = s_ref[...]
    h = jnp.dot(s, w1_ref[...], preferred_element_type=jnp.float32) + b1_ref[...]
    h = h * jax.nn.sigmoid(h)
    phi = jnp.dot(h, w2_ref[...], preferred_element_type=jnp.float32) + b2_ref[...]
    lo_ref[...] = phi[:, :F]
    hi_ref[...] = phi[:, F:]


def _node_mlp(s_j, W1, b1, W2, b2):
    n = s_j.shape[0]
    bn = 2000
    return pl.pallas_call(
        _mlp_body,
        grid=(n // bn,),
        in_specs=[
            pl.BlockSpec((bn, F), lambda i: (i, 0)),
            pl.BlockSpec((F, F), lambda i: (0, 0)),
            pl.BlockSpec((1, F), lambda i: (0, 0)),
            pl.BlockSpec((F, 2 * F), lambda i: (0, 0)),
            pl.BlockSpec((1, 2 * F), lambda i: (0, 0)),
        ],
        out_specs=[
            pl.BlockSpec((bn, F), lambda i: (i, 0)),
            pl.BlockSpec((bn, F), lambda i: (i, 0)),
        ],
        out_shape=[jax.ShapeDtypeStruct((n, F), jnp.float32)] * 2,
    )(s_j, W1, b1.reshape(1, F), W2, b2.reshape(1, 2 * F))


def _edge_body(r_ref, wr_ref, br_ref, cs_ref, cv0_ref, cv1_ref, cv2_ref):
    r = r_ref[...]
    d2 = jnp.sum(r * r + 1e-8, axis=1, keepdims=True)
    dist = jnp.sqrt(d2)
    unit = r / dist
    ni = lax.broadcasted_iota(jnp.int32, (1, N_RBF + 1), 1)
    nf = ni.astype(jnp.float32) + 1.0
    coef = jnp.where(ni < N_RBF, nf * (jnp.pi / CUTOFF),
                     jnp.pi / (2.0 * CUTOFF))
    s = _fast_sin(dist * coef)
    denom = jnp.where(dist == 0.0, 1.0, dist)
    rbf = jnp.where(dist >= CUTOFF, 0.0, s[:, :N_RBF] / denom)
    sh = s[:, N_RBF:N_RBF + 1]
    env = jnp.where(dist <= CUTOFF, 1.0 - sh * sh, 0.0)
    w = jnp.dot(rbf, wr_ref[...], preferred_element_type=jnp.float32)
    w = (w + br_ref[...]) * env
    w_lo = w[:, :F]
    cs_ref[...] = _pack_half(w[:, F:])
    cv0_ref[...] = _pack_half(w_lo * unit[:, 0:1])
    cv1_ref[...] = _pack_half(w_lo * unit[:, 1:2])
    cv2_ref[...] = _pack_half(w_lo * unit[:, 2:3])


_SIN_C = (0.9999999959621529, -0.1666666504215155, 0.008333314504480197,
          -0.00019840310898311397, 2.753228835635207e-06,
          -2.470157612483037e-08, 1.3533146825395193e-10)


def _fast_sin(x):
    k = jnp.round(x * (1.0 / (2.0 * jnp.pi)))
    y = x - k * (2.0 * jnp.pi)
    t = y * y
    acc = jnp.float32(_SIN_C[6])
    for c in _SIN_C[5::-1]:
        acc = acc * t + jnp.float32(c)
    return y * acc


def _pack_half(c):
    half = c.shape[0] // 2
    bits = lax.bitcast_convert_type(c, jnp.uint32)
    lo = (bits[:half] + jnp.uint32(0x8000)) >> 16
    hi = (bits[half:] + jnp.uint32(0x8000)) & jnp.uint32(0xFFFF0000)
    return lo | hi


def _edge_coeffs(r_ij, Wr, br):
    e = r_ij.shape[0]
    be = 2000
    return pl.pallas_call(
        _edge_body,
        grid=(e // be,),
        in_specs=[
            pl.BlockSpec((be, 3), lambda i: (i, 0)),
            pl.BlockSpec((N_RBF, 2 * F), lambda i: (0, 0)),
            pl.BlockSpec((1, 2 * F), lambda i: (0, 0)),
        ],
        out_specs=[pl.BlockSpec((be // 2, F), lambda i: (i, 0))] * 4,
        out_shape=[jax.ShapeDtypeStruct((e // 2, F), jnp.uint32)] * 4,
    )(r_ij, Wr, br.reshape(1, 2 * F))


def _make_sc_pass(n, e):
    epw = e // NW
    B = B_EDGE
    nb = epw // B
    stripe = 640
    chunks = stripe // B
    mesh = plsc.VectorSubcoreMesh(core_axis_name="c", subcore_axis_name="s")

    @functools.partial(
        pl.kernel,
        mesh=mesh,
        out_type=jax.ShapeDtypeStruct((NC, n, F), jnp.float32),
        scratch_types=[
            pltpu.VMEM((2, B), jnp.int32),
            pltpu.VMEM((2, B), jnp.int32),
            pltpu.VMEM((2, B), jnp.int32),
            pltpu.VMEM((2, B, F), jnp.float32),
            pltpu.VMEM((2, B // 2, F), jnp.uint32),
            pltpu.VMEM_SHARED((n, F), jnp.float32),
            pltpu.SemaphoreType.DMA,
            pltpu.SemaphoreType.DMA,
            pltpu.SemaphoreType.DMA,
            pltpu.SemaphoreType.DMA,
            pltpu.SemaphoreType.DMA,
            pltpu.SemaphoreType.DMA,
            pltpu.SemaphoreType.DMA,
            pltpu.SemaphoreType.DMA,
        ],
    )
    def sc_pass(phi_hbm, c_hbm, src_hbm, dst_hbm, out_hbm,
                src_v, dst_v, dstx_v, rows_v, c_v, slab,
                g0, g1, c0, c1, i0, i1, s0, s1):
        cid = lax.axis_index("c")
        sid = lax.axis_index("s")
        wid = sid * NC + cid
        row0 = sid * stripe
        my_rows = jnp.where(sid == NS - 1, n - (NS - 1) * stripe, stripe)
        gsem = (g0, g1)
        csem = (c0, c1)
        isem = (i0, i1)
        ssem = (s0, s1)

        def zbody(i, carry):
            for j in range(8):
                rows_v[0, i, pl.ds(j * 16, 16)] = jnp.zeros((16,), jnp.float32)
            return carry
        lax.fori_loop(0, B, zbody, 0)
        for q in range(chunks):
            @pl.when(q * B < my_rows)
            def _():
                pltpu.sync_copy(rows_v.at[0],
                                slab.at[pl.ds(row0 + q * B, B), :])
        plsc.subcore_barrier()

        def issue_idx(g, slot):
            base = wid * epw + g * B
            pltpu.async_copy(src_hbm.at[pl.ds(base, B)], src_v.at[slot],
                             isem[slot])
            pltpu.async_copy(dst_hbm.at[pl.ds(base, B)], dst_v.at[slot],
                             isem[slot])

        def wait_idx(g, slot):
            base = wid * epw + g * B
            pltpu.make_async_copy(src_hbm.at[pl.ds(base, B)], src_v.at[slot],
                                  isem[slot]).wait()
            pltpu.make_async_copy(dst_hbm.at[pl.ds(base, B)], dst_v.at[slot],
                                  isem[slot]).wait()

        def issue_data(g, slot):
            pltpu.async_copy(phi_hbm.at[src_v.at[slot]], rows_v.at[slot],
                             gsem[slot])
            cbase = wid * (epw // 2) + g * (B // 2)
            pltpu.async_copy(c_hbm.at[pl.ds(cbase, B // 2), :], c_v.at[slot],
                             csem[slot])

        issue_idx(0, 0)
        wait_idx(0, 0)
        issue_idx(1, 1)
        issue_data(0, 0)

        def pair(i, carry):
            for b in range(2):
                g = 2 * i + b
                slot = b
                nslot = 1 - b

                @pl.when(g < nb)
                def _():
                    pltpu.make_async_copy(phi_hbm.at[src_v.at[slot]],
                                          rows_v.at[slot], gsem[slot]).wait()

                    @pl.when(g + 1 < nb)
                    def _():
                        wait_idx(g + 1, nslot)

                        @pl.when(g >= 1)
                        def _():
                            pltpu.make_async_copy(
                                rows_v.at[nslot], slab.at[dstx_v.at[nslot]],
                                ssem[nslot]).wait()
                        issue_data(g + 1, nslot)

                    cbase = wid * (epw // 2) + g * (B // 2)
                    pltpu.make_async_copy(c_hbm.at[pl.ds(cbase, B // 2), :],
                                          c_v.at[slot], csem[slot]).wait()

                    mask = jnp.full((16,), 0xFFFF0000, jnp.uint32)

                    def bc(x):
                        return lax.bitcast_convert_type(x, jnp.float32)

                    def mul(ep, c2):
                        e2 = B // 2 + ep
                        for j in range(8):
                            sl = pl.ds(j * 16, 16)
                            cw = c_v[slot, ep, sl]
                            rows_v[slot, ep, sl] = (rows_v[slot, ep, sl]
                                                    * bc(cw << 16))
                            rows_v[slot, e2, sl] = (rows_v[slot, e2, sl]
                                                    * bc(cw & mask))
                        return c2
                    lax.fori_loop(0, B // 2, mul, 0)

                    def icpy(q, c2):
                        sl = pl.ds(q * 16, 16)
                        dstx_v[slot, sl] = dst_v[slot, sl]
                        return c2
                    lax.fori_loop(0, B // 16, icpy, 0)
                    pltpu.async_copy(rows_v.at[slot],
                                     slab.at[dstx_v.at[slot]], ssem[slot],
                                     add=True)

                    @pl.when(g + 2 < nb)
                    def _():
                        issue_idx(g + 2, slot)
            return carry
        lax.fori_loop(0, (nb + 1) // 2, pair, 0)

        for s in range(2):
            pltpu.make_async_copy(rows_v.at[s], slab.at[dstx_v.at[s]],
                                  ssem[s]).wait()

        plsc.subcore_barrier()
        for q in range(chunks):
            @pl.when(q * B < my_rows)
            def _():
                pltpu.sync_copy(slab.at[pl.ds(row0 + q * B, B), :],
                                out_hbm.at[cid, pl.ds(row0 + q * B, B), :])

    return sc_pass


def _sum_body(ps_ref, p0_ref, p1_ref, p2_ref, s_ref, v0_ref, v1_ref, v2_ref):
    s_ref[...] = ps_ref[0] + ps_ref[1]
    v0_ref[...] = p0_ref[0] + p0_ref[1]
    v1_ref[...] = p1_ref[0] + p1_ref[1]
    v2_ref[...] = p2_ref[0] + p2_ref[1]


def _sum_partials(ps, p0, p1, p2):
    n = ps.shape[1]
    bn = 2000
    return pl.pallas_call(
        _sum_body,
        grid=(n // bn,),
        in_specs=[pl.BlockSpec((NC, bn, F), lambda i: (0, i, 0))] * 4,
        out_specs=[pl.BlockSpec((bn, F), lambda i: (i, 0))] * 4,
        out_shape=[jax.ShapeDtypeStruct((n, F), jnp.float32)] * 4,
    )(ps, p0, p1, p2)


def kernel(s_j, v_j, r_ij, nbrs, W1, b1, W2, b2, Wr, br):
    del v_j
    n = s_j.shape[0]
    e = r_ij.shape[0]

    phi_lo, phi_hi = _node_mlp(s_j, W1, b1, W2, b2)
    c_s, c_v0, c_v1, c_v2 = _edge_coeffs(r_ij, Wr, br)

    def _pair_order(col):
        return (col.astype(jnp.int32)
                .reshape(e // 2000, 2, 25, 40)
                .transpose(0, 2, 1, 3)
                .reshape(-1))

    dst = _pair_order(nbrs[:, 0])
    src = _pair_order(nbrs[:, 1])

    sc_pass = _make_sc_pass(n, e)
    ps = sc_pass(phi_hi, c_s, src, dst)
    pv0 = sc_pass(phi_lo, c_v0, src, dst)
    pv1 = sc_pass(phi_lo, c_v1, src, dst)
    pv2 = sc_pass(phi_lo, c_v2, src, dst)

    delta_s, v0, v1, v2 = _sum_partials(ps, pv0, pv1, pv2)
    delta_v = jnp.stack([v0, v1, v2], axis=-1)
    return (delta_s, delta_v)
